# baseline (speedup 1.0000x reference)
import jax
import jax.numpy as jnp
from jax.experimental import pallas as pl
from jax.experimental.pallas import tpu as pltpu

B, N, DIN, DOUT = 4, 2048, 128, 128
BM = 512
NSPLIT = 4
BMC = BM // NSPLIT


def _gcn_body(x_ref, w_ref, a0, a1, a2, a3, bias_ref, out_ref):
    j = pl.program_id(1)

    sup = jnp.dot(
        x_ref[0], w_ref[...], preferred_element_type=jnp.float32
    ).astype(jnp.bfloat16)

    partial = None
    for i, a in enumerate((a0, a1, a2, a3)):
        p = jax.lax.dot_general(
            a[0].astype(jnp.bfloat16),
            sup[i * BMC:(i + 1) * BMC, :],
            (((0,), (0,)), ((), ())),
            preferred_element_type=jnp.float32,
        )
        partial = p if partial is None else partial + p

    @pl.when(j == 0)
    def _():
        out_ref[0] = partial + bias_ref[...]

    @pl.when(j != 0)
    def _():
        out_ref[0] += partial


@jax.jit
def kernel(input, adj, weight, bias):
    bias2d = bias.reshape(1, DOUT)
    grid = (B, N // BM)

    def chunk_spec(i):
        return pl.BlockSpec((1, BMC, N), lambda b, j, i=i: (b, NSPLIT * j + i, 0))

    return pl.pallas_call(
        _gcn_body,
        grid=grid,
        in_specs=[
            pl.BlockSpec((1, BM, DIN), lambda b, j: (b, j, 0)),
            pl.BlockSpec((DIN, DOUT), lambda b, j: (0, 0)),
            chunk_spec(0),
            chunk_spec(1),
            chunk_spec(2),
            chunk_spec(3),
            pl.BlockSpec((1, DOUT), lambda b, j: (0, 0)),
        ],
        out_specs=pl.BlockSpec((1, N, DOUT), lambda b, j: (b, 0, 0)),
        out_shape=jax.ShapeDtypeStruct((B, N, DOUT), jnp.float32),
        compiler_params=pltpu.CompilerParams(
            dimension_semantics=("arbitrary", "arbitrary"),
        ),
    )(input, weight, adj, adj, adj, adj, bias2d)


# speedup vs baseline: 1.1118x; 1.1118x over previous
import jax
import jax.numpy as jnp
from jax.experimental import pallas as pl
from jax.experimental.pallas import tpu as pltpu

B, N, DIN, DOUT = 4, 2048, 128, 128
GBM = 512
NG = N // GBM
TOTAL = B * NG
NCH = 4
CH = GBM // NCH
NSLOT = 3


def _gcn_body(x_ref, w_ref, adj_hbm, bias_ref, out_ref, sup_ref, abuf, sems):
    b = pl.program_id(0)
    g = pl.program_id(1)
    step = b * NG + g

    @pl.when(g == 0)
    def _():
        sup_ref[...] = jnp.dot(
            x_ref[0], w_ref[...], preferred_element_type=jnp.float32
        ).astype(jnp.bfloat16)

    def copy(k, i):
        kb = k // NG
        kg = k % NG
        return pltpu.make_async_copy(
            adj_hbm.at[kb, pl.ds(kg * GBM + i * CH, CH), :],
            abuf.at[k % NSLOT, pl.ds(i * CH, CH), :],
            sems.at[k % NSLOT, i],
        )

    @pl.when(step == 0)
    def _():
        for i in range(NCH):
            copy(0, i).start()
        for i in range(NCH):
            copy(1, i).start()

    @pl.when(step + 2 < TOTAL)
    def _():
        for i in range(NCH):
            copy(step + 2, i).start()

    for i in range(NCH):
        copy(step, i).wait()

    partial = jax.lax.dot_general(
        abuf[step % NSLOT].astype(jnp.bfloat16),
        sup_ref[pl.ds(g * GBM, GBM), :],
        (((0,), (0,)), ((), ())),
        preferred_element_type=jnp.float32,
    )

    @pl.when(g == 0)
    def _():
        out_ref[0] = partial + bias_ref[...]

    @pl.when(g != 0)
    def _():
        out_ref[0] += partial


@jax.jit
def kernel(input, adj, weight, bias):
    bias2d = bias.reshape(1, DOUT)
    grid = (B, NG)
    return pl.pallas_call(
        _gcn_body,
        grid=grid,
        in_specs=[
            pl.BlockSpec((1, N, DIN), lambda b, g: (b, 0, 0)),
            pl.BlockSpec((DIN, DOUT), lambda b, g: (0, 0)),
            pl.BlockSpec(memory_space=pl.ANY),
            pl.BlockSpec((1, DOUT), lambda b, g: (0, 0)),
        ],
        out_specs=pl.BlockSpec((1, N, DOUT), lambda b, g: (b, 0, 0)),
        out_shape=jax.ShapeDtypeStruct((B, N, DOUT), jnp.float32),
        scratch_shapes=[
            pltpu.VMEM((N, DOUT), jnp.bfloat16),
            pltpu.VMEM((NSLOT, GBM, N), jnp.float32),
            pltpu.SemaphoreType.DMA((NSLOT, NCH)),
        ],
        compiler_params=pltpu.CompilerParams(
            dimension_semantics=("arbitrary", "arbitrary"),
        ),
    )(input, weight, adj, bias2d)
